# baseline (speedup 1.0000x reference)
import functools

import jax
import jax.numpy as jnp
from jax import lax
from jax.experimental import pallas as pl
from jax.experimental.pallas import tpu as pltpu
from jax.experimental.pallas import tpu_sc as plsc

N = 2048
D = 1024
E = 8
F = 512
SH = 1024
EPS = 1e-20
BLK = 256
NT = 24
L = NT * BLK
RB = 128
NC, NS = 2, 16
NW = NC * NS
TPW = N // NW

_HI = lax.Precision.HIGHEST


def _sigmoid(v):
    return 1.0 / (1.0 + jnp.exp(-v))


def _router_body(x_ref, rw_ref, p0_ref, p1_ref, w_ref, te_ref, va_ref):
    xf = x_ref[...]
    rwt = rw_ref[...]
    logits = lax.dot_general(xf, rwt, (((1,), (0,)), ((), ())),
                             preferred_element_type=jnp.float32)
    lt = logits.T
    m = jnp.max(lt, axis=0, keepdims=True)
    eg = jnp.exp(lt - m)
    gates = eg / jnp.sum(eg, axis=0, keepdims=True)

    iota_e = lax.broadcasted_iota(jnp.int32, (E, N), 0)
    v1 = jnp.max(gates, axis=0, keepdims=True)
    i1 = jnp.min(jnp.where(gates == v1, iota_e, E), axis=0, keepdims=True)
    g2 = jnp.where(iota_e == i1, -1.0, gates)
    v2 = jnp.max(g2, axis=0, keepdims=True)
    i2 = jnp.min(jnp.where(g2 == v2, iota_e, E), axis=0, keepdims=True)
    denom = v1 + v2 + EPS
    w01 = jnp.concatenate([v1 / denom, v2 / denom], axis=0)
    w_ref[...] = w01.T

    i1g = i1.reshape(N // RB, RB)
    i2g = i2.reshape(N // RB, RB)
    a = lax.broadcasted_iota(jnp.int32, (RB, RB), 0)
    b = lax.broadcasted_iota(jnp.int32, (RB, RB), 1)
    u_incl = (a <= b).astype(jnp.float32)
    r32 = N // RB * 2
    c = lax.broadcasted_iota(jnp.int32, (r32, r32), 0)
    d = lax.broadcasted_iota(jnp.int32, (r32, r32), 1)
    s_strict = (d < c).astype(jnp.float32)

    iota_t = lax.broadcasted_iota(jnp.int32, (1, RB), 1)
    p_acc = jnp.zeros((r32, RB), jnp.float32)
    te_acc = jnp.zeros((1, RB), jnp.int32)
    va_acc = jnp.zeros((1, RB), jnp.int32)
    pstart = jnp.float32(0.0)
    for e in range(E):
        me = jnp.concatenate([(i1g == e), (i2g == e)], axis=0).astype(jnp.float32)
        incl = lax.dot_general(me, u_incl, (((1,), (0,)), ((), ())),
                               precision=_HI, preferred_element_type=jnp.float32)
        excl = incl - me
        row_tot = incl[:, RB - 1:RB]
        row_off = lax.dot_general(s_strict, row_tot, (((1,), (0,)), ((), ())),
                                  precision=_HI, preferred_element_type=jnp.float32)
        rank = excl + row_off
        cnt = row_off[r32 - 1, 0] + row_tot[r32 - 1, 0]
        ntile = jnp.floor((cnt + (BLK - 1)) * (1.0 / BLK))
        p_acc = p_acc + me * (pstart + rank)
        t0 = (pstart * (1.0 / BLK)).astype(jnp.int32)
        t1 = t0 + ntile.astype(jnp.int32)
        mask_t = jnp.logical_and(iota_t >= t0, iota_t < t1)
        te_acc = te_acc + jnp.where(mask_t, e, 0)
        va_acc = va_acc + jnp.where(mask_t, 1, 0)
        pstart = pstart + ntile * BLK

    p = p_acc.astype(jnp.int32)
    p0_ref[...] = p[:N // RB]
    p1_ref[...] = p[N // RB:]
    te_ref[...] = te_acc
    va_ref[...] = va_acc


def _router_dispatch(xf, router_w, interpret=False):
    return pl.pallas_call(
        _router_body,
        out_shape=(
            jax.ShapeDtypeStruct((N // RB, RB), jnp.int32),
            jax.ShapeDtypeStruct((N // RB, RB), jnp.int32),
            jax.ShapeDtypeStruct((N, 2), jnp.float32),
            jax.ShapeDtypeStruct((1, RB), jnp.int32),
            jax.ShapeDtypeStruct((1, RB), jnp.int32),
        ),
        interpret=interpret,
    )(xf, router_w)


def _sc_dispatch_body(xf_hbm, p0_hbm, p1_hbm, xg_hbm, idxa_v, idxb_v, rows_v,
                      sema, semb):
    wid = lax.axis_index("s") * NC + lax.axis_index("c")
    base = wid * TPW
    pltpu.sync_copy(xf_hbm.at[pl.ds(base, TPW)], rows_v)
    pltpu.sync_copy(p0_hbm.at[pl.ds(base, TPW)], idxa_v)
    pltpu.sync_copy(p1_hbm.at[pl.ds(base, TPW)], idxb_v)
    ca = pltpu.async_copy(rows_v, xg_hbm.at[idxa_v], sema)
    cb = pltpu.async_copy(rows_v, xg_hbm.at[idxb_v], semb)
    ca.wait()
    cb.wait()


def _sc_dispatch(xf, p0, p1):
    mesh = plsc.VectorSubcoreMesh(core_axis_name="c", subcore_axis_name="s")
    return pl.kernel(
        _sc_dispatch_body,
        out_type=jax.ShapeDtypeStruct((L, D), jnp.float32),
        mesh=mesh,
        scratch_types=[
            pltpu.VMEM((TPW,), jnp.int32),
            pltpu.VMEM((TPW,), jnp.int32),
            pltpu.VMEM((TPW, D), jnp.float32),
            pltpu.SemaphoreType.DMA,
            pltpu.SemaphoreType.DMA,
        ],
    )(xf, p0, p1)


def _gffn_body(te_ref, va_ref, xg_ref, eg_ref, eu_ref, ed_ref, y_ref):
    t = pl.program_id(0)

    @pl.when(va_ref[0, t] == 1)
    def _compute():
        e = te_ref[0, t]
        xt = xg_ref[...]
        hg = lax.dot_general(xt, eg_ref[e], (((1,), (1,)), ((), ())),
                             preferred_element_type=jnp.float32)
        hu = lax.dot_general(xt, eu_ref[e], (((1,), (1,)), ((), ())),
                             preferred_element_type=jnp.float32)
        h = hg * _sigmoid(hg) * hu
        y_ref[...] = lax.dot_general(h, ed_ref[e], (((1,), (1,)), ((), ())),
                                     preferred_element_type=jnp.float32)


def _grouped_ffn(te, va, xg, eg, eu, ed, interpret=False):
    single = pl.Buffered(buffer_count=1)
    return pl.pallas_call(
        _gffn_body,
        grid=(NT,),
        in_specs=[
            pl.BlockSpec(memory_space=pltpu.SMEM),
            pl.BlockSpec(memory_space=pltpu.SMEM),
            pl.BlockSpec((BLK, D), lambda t: (t, 0)),
            pl.BlockSpec((E, F, D), lambda t: (0, 0, 0), pipeline_mode=single),
            pl.BlockSpec((E, F, D), lambda t: (0, 0, 0), pipeline_mode=single),
            pl.BlockSpec((E, D, F), lambda t: (0, 0, 0), pipeline_mode=single),
        ],
        out_specs=pl.BlockSpec((BLK, D), lambda t: (t, 0)),
        out_shape=jax.ShapeDtypeStruct((L, D), jnp.float32),
        interpret=interpret,
    )(te, va, xg, eg, eu, ed)


def _shared_body(x_ref, sg_ref, su_ref, sd_ref, w_ref, y0_ref, y1_ref, o_ref):
    xt = x_ref[...]
    hg = lax.dot_general(xt, sg_ref[...], (((1,), (1,)), ((), ())),
                         preferred_element_type=jnp.float32)
    hu = lax.dot_general(xt, su_ref[...], (((1,), (1,)), ((), ())),
                         preferred_element_type=jnp.float32)
    h = hg * _sigmoid(hg) * hu
    sh = lax.dot_general(h, sd_ref[...], (((1,), (1,)), ((), ())),
                         preferred_element_type=jnp.float32)
    w0 = w_ref[:, 0:1]
    w1 = w_ref[:, 1:2]
    o_ref[...] = w0 * y0_ref[...] + w1 * y1_ref[...] + sh


def _shared_combine(xf, sg, su, sd, w, y0, y1, interpret=False):
    nt = N // BLK
    return pl.pallas_call(
        _shared_body,
        grid=(nt,),
        in_specs=[
            pl.BlockSpec((BLK, D), lambda t: (t, 0)),
            pl.BlockSpec((SH, D), lambda t: (0, 0)),
            pl.BlockSpec((SH, D), lambda t: (0, 0)),
            pl.BlockSpec((D, SH), lambda t: (0, 0)),
            pl.BlockSpec((BLK, 2), lambda t: (t, 0)),
            pl.BlockSpec((BLK, D), lambda t: (t, 0)),
            pl.BlockSpec((BLK, D), lambda t: (t, 0)),
        ],
        out_specs=pl.BlockSpec((BLK, D), lambda t: (t, 0)),
        out_shape=jax.ShapeDtypeStruct((N, D), jnp.float32),
        interpret=interpret,
    )(xf, sg, su, sd, w, y0, y1)


def _sc_combine_body(yg_hbm, p0_hbm, p1_hbm, y0_hbm, y1_hbm,
                     idxa_v, idxb_v, rowsa_v, rowsb_v, sema, semb):
    wid = lax.axis_index("s") * NC + lax.axis_index("c")
    base = wid * TPW
    half = TPW // 2
    for hh in range(2):
        b2 = base + hh * half
        pltpu.sync_copy(p0_hbm.at[pl.ds(b2, half)], idxa_v)
        pltpu.sync_copy(p1_hbm.at[pl.ds(b2, half)], idxb_v)
        ca = pltpu.async_copy(yg_hbm.at[idxa_v], rowsa_v, sema)
        cb = pltpu.async_copy(yg_hbm.at[idxb_v], rowsb_v, semb)
        ca.wait()
        pltpu.sync_copy(rowsa_v, y0_hbm.at[pl.ds(b2, half)])
        cb.wait()
        pltpu.sync_copy(rowsb_v, y1_hbm.at[pl.ds(b2, half)])


def _sc_combine(yg, p0, p1):
    mesh = plsc.VectorSubcoreMesh(core_axis_name="c", subcore_axis_name="s")
    return pl.kernel(
        _sc_combine_body,
        out_type=(
            jax.ShapeDtypeStruct((N, D), jnp.float32),
            jax.ShapeDtypeStruct((N, D), jnp.float32),
        ),
        mesh=mesh,
        scratch_types=[
            pltpu.VMEM((TPW // 2,), jnp.int32),
            pltpu.VMEM((TPW // 2,), jnp.int32),
            pltpu.VMEM((TPW // 2, D), jnp.float32),
            pltpu.VMEM((TPW // 2, D), jnp.float32),
            pltpu.SemaphoreType.DMA,
            pltpu.SemaphoreType.DMA,
        ],
    )(yg, p0, p1)


def kernel(x, router_w, expert_gate, expert_up, expert_down,
           shared_gate, shared_up, shared_down):
    xf = x.reshape(N, D)
    p0r, p1r, w, te, va = _router_dispatch(xf, router_w.T)
    p0 = p0r.reshape(N)
    p1 = p1r.reshape(N)
    xg = _sc_dispatch(xf, p0, p1)
    yg = _grouped_ffn(te, va, xg, expert_gate, expert_up, expert_down)
    y0, y1 = _sc_combine(yg, p0, p1)
    out = _shared_combine(xf, shared_gate, shared_up, shared_down, w, y0, y1)
    return out.reshape(1, N, D)


# speedup vs baseline: 1.0159x; 1.0159x over previous
import functools

import jax
import jax.numpy as jnp
from jax import lax
from jax.experimental import pallas as pl
from jax.experimental.pallas import tpu as pltpu
from jax.experimental.pallas import tpu_sc as plsc

N = 2048
D = 1024
E = 8
F = 512
SH = 1024
EPS = 1e-20
BLK = 256
NT = 24
L = NT * BLK
RB = 128
NC, NS = 2, 16
NW = NC * NS
TPW = N // NW

_HI = lax.Precision.HIGHEST


def _sigmoid(v):
    return 1.0 / (1.0 + jnp.exp(-v))


def _router_body(x_ref, rw_ref, p0_ref, p1_ref, w_ref, te_ref, va_ref):
    xf = x_ref[...]
    rwt = rw_ref[...]
    logits = lax.dot_general(xf, rwt, (((1,), (0,)), ((), ())),
                             preferred_element_type=jnp.float32)
    lt = logits.T
    m = jnp.max(lt, axis=0, keepdims=True)
    eg = jnp.exp(lt - m)
    gates = eg / jnp.sum(eg, axis=0, keepdims=True)

    iota_e = lax.broadcasted_iota(jnp.int32, (E, N), 0)
    v1 = jnp.max(gates, axis=0, keepdims=True)
    i1 = jnp.min(jnp.where(gates == v1, iota_e, E), axis=0, keepdims=True)
    g2 = jnp.where(iota_e == i1, -1.0, gates)
    v2 = jnp.max(g2, axis=0, keepdims=True)
    i2 = jnp.min(jnp.where(g2 == v2, iota_e, E), axis=0, keepdims=True)
    denom = v1 + v2 + EPS
    w01 = jnp.concatenate([v1 / denom, v2 / denom], axis=0)
    w_ref[...] = w01.T

    i1g = i1.reshape(N // RB, RB)
    i2g = i2.reshape(N // RB, RB)
    a = lax.broadcasted_iota(jnp.int32, (RB, RB), 0)
    b = lax.broadcasted_iota(jnp.int32, (RB, RB), 1)
    u_incl = (a <= b).astype(jnp.float32)
    r32 = N // RB * 2
    c = lax.broadcasted_iota(jnp.int32, (r32, r32), 0)
    d = lax.broadcasted_iota(jnp.int32, (r32, r32), 1)
    s_strict = (d < c).astype(jnp.float32)

    iota_t = lax.broadcasted_iota(jnp.int32, (1, RB), 1)
    p_acc = jnp.zeros((r32, RB), jnp.float32)
    te_acc = jnp.zeros((1, RB), jnp.int32)
    va_acc = jnp.zeros((1, RB), jnp.int32)
    pstart = jnp.float32(0.0)
    for e in range(E):
        me = jnp.concatenate([(i1g == e), (i2g == e)], axis=0).astype(jnp.float32)
        incl = lax.dot_general(me, u_incl, (((1,), (0,)), ((), ())),
                               precision=_HI, preferred_element_type=jnp.float32)
        excl = incl - me
        row_tot = incl[:, RB - 1:RB]
        row_off = lax.dot_general(s_strict, row_tot, (((1,), (0,)), ((), ())),
                                  precision=_HI, preferred_element_type=jnp.float32)
        rank = excl + row_off
        cnt = row_off[r32 - 1, 0] + row_tot[r32 - 1, 0]
        ntile = jnp.floor((cnt + (BLK - 1)) * (1.0 / BLK))
        p_acc = p_acc + me * (pstart + rank)
        t0 = (pstart * (1.0 / BLK)).astype(jnp.int32)
        t1 = t0 + ntile.astype(jnp.int32)
        mask_t = jnp.logical_and(iota_t >= t0, iota_t < t1)
        te_acc = te_acc + jnp.where(mask_t, e, 0)
        va_acc = va_acc + jnp.where(mask_t, 1, 0)
        pstart = pstart + ntile * BLK

    p = p_acc.astype(jnp.int32)
    p0_ref[...] = p[:N // RB]
    p1_ref[...] = p[N // RB:]
    te_ref[...] = te_acc
    va_ref[...] = va_acc


def _router_dispatch(xf, router_w, interpret=False):
    return pl.pallas_call(
        _router_body,
        out_shape=(
            jax.ShapeDtypeStruct((N // RB, RB), jnp.int32),
            jax.ShapeDtypeStruct((N // RB, RB), jnp.int32),
            jax.ShapeDtypeStruct((N, 2), jnp.float32),
            jax.ShapeDtypeStruct((1, RB), jnp.int32),
            jax.ShapeDtypeStruct((1, RB), jnp.int32),
        ),
        interpret=interpret,
    )(xf, router_w)


def _sc_dispatch_body(xf_hbm, p0_hbm, p1_hbm, xg_hbm, idx_v, rows_v):
    wid = lax.axis_index("s") * NC + lax.axis_index("c")
    base = wid * TPW
    pltpu.sync_copy(xf_hbm.at[pl.ds(base, TPW)], rows_v)
    pltpu.sync_copy(p0_hbm.at[pl.ds(base, TPW)], idx_v)
    pltpu.sync_copy(rows_v, xg_hbm.at[idx_v])
    pltpu.sync_copy(p1_hbm.at[pl.ds(base, TPW)], idx_v)
    pltpu.sync_copy(rows_v, xg_hbm.at[idx_v])


def _sc_dispatch(xf, p0, p1):
    mesh = plsc.VectorSubcoreMesh(core_axis_name="c", subcore_axis_name="s")
    return pl.kernel(
        _sc_dispatch_body,
        out_type=jax.ShapeDtypeStruct((L, D), jnp.float32),
        mesh=mesh,
        scratch_types=[
            pltpu.VMEM((TPW,), jnp.int32),
            pltpu.VMEM((TPW, D), jnp.float32),
        ],
    )(xf, p0, p1)


def _gffn_body(te_ref, va_ref, xg_ref, eg_ref, eu_ref, ed_ref, y_ref):
    t = pl.program_id(0)

    @pl.when(va_ref[0, t] == 1)
    def _compute():
        e = te_ref[0, t]
        xt = xg_ref[...]
        hg = lax.dot_general(xt, eg_ref[e], (((1,), (1,)), ((), ())),
                             preferred_element_type=jnp.float32)
        hu = lax.dot_general(xt, eu_ref[e], (((1,), (1,)), ((), ())),
                             preferred_element_type=jnp.float32)
        h = hg * _sigmoid(hg) * hu
        y_ref[...] = lax.dot_general(h, ed_ref[e], (((1,), (1,)), ((), ())),
                                     preferred_element_type=jnp.float32)


def _grouped_ffn(te, va, xg, eg, eu, ed, interpret=False):
    single = pl.Buffered(buffer_count=1)
    return pl.pallas_call(
        _gffn_body,
        grid=(NT,),
        in_specs=[
            pl.BlockSpec(memory_space=pltpu.SMEM),
            pl.BlockSpec(memory_space=pltpu.SMEM),
            pl.BlockSpec((BLK, D), lambda t: (t, 0)),
            pl.BlockSpec((E, F, D), lambda t: (0, 0, 0), pipeline_mode=single),
            pl.BlockSpec((E, F, D), lambda t: (0, 0, 0), pipeline_mode=single),
            pl.BlockSpec((E, D, F), lambda t: (0, 0, 0), pipeline_mode=single),
        ],
        out_specs=pl.BlockSpec((BLK, D), lambda t: (t, 0)),
        out_shape=jax.ShapeDtypeStruct((L, D), jnp.float32),
        interpret=interpret,
    )(te, va, xg, eg, eu, ed)


def _shared_body(x_ref, sg_ref, su_ref, sd_ref, o_ref):
    xt = x_ref[...]
    hg = lax.dot_general(xt, sg_ref[...], (((1,), (1,)), ((), ())),
                         preferred_element_type=jnp.float32)
    hu = lax.dot_general(xt, su_ref[...], (((1,), (1,)), ((), ())),
                         preferred_element_type=jnp.float32)
    h = hg * _sigmoid(hg) * hu
    o_ref[...] = lax.dot_general(h, sd_ref[...], (((1,), (1,)), ((), ())),
                                 preferred_element_type=jnp.float32)


def _shared_ffn(xf, sg, su, sd, interpret=False):
    nt = N // BLK
    return pl.pallas_call(
        _shared_body,
        grid=(nt,),
        in_specs=[
            pl.BlockSpec((BLK, D), lambda t: (t, 0)),
            pl.BlockSpec((SH, D), lambda t: (0, 0)),
            pl.BlockSpec((SH, D), lambda t: (0, 0)),
            pl.BlockSpec((D, SH), lambda t: (0, 0)),
        ],
        out_specs=pl.BlockSpec((BLK, D), lambda t: (t, 0)),
        out_shape=jax.ShapeDtypeStruct((N, D), jnp.float32),
        interpret=interpret,
    )(xf, sg, su, sd)


def _sc_combine_body(yg_hbm, p0_hbm, p1_hbm, y0_hbm, y1_hbm, idx_v, rows_v, sem):
    wid = lax.axis_index("s") * NC + lax.axis_index("c")
    base = wid * TPW
    pltpu.sync_copy(p0_hbm.at[pl.ds(base, TPW)], idx_v)
    pltpu.async_copy(yg_hbm.at[idx_v], rows_v, sem).wait()
    pltpu.sync_copy(rows_v, y0_hbm.at[pl.ds(base, TPW)])
    pltpu.sync_copy(p1_hbm.at[pl.ds(base, TPW)], idx_v)
    pltpu.async_copy(yg_hbm.at[idx_v], rows_v, sem).wait()
    pltpu.sync_copy(rows_v, y1_hbm.at[pl.ds(base, TPW)])


def _sc_combine(yg, p0, p1):
    mesh = plsc.VectorSubcoreMesh(core_axis_name="c", subcore_axis_name="s")
    return pl.kernel(
        _sc_combine_body,
        out_type=(
            jax.ShapeDtypeStruct((N, D), jnp.float32),
            jax.ShapeDtypeStruct((N, D), jnp.float32),
        ),
        mesh=mesh,
        scratch_types=[
            pltpu.VMEM((TPW,), jnp.int32),
            pltpu.VMEM((TPW, D), jnp.float32),
            pltpu.SemaphoreType.DMA,
        ],
    )(yg, p0, p1)


def _final_body(w_ref, y0_ref, y1_ref, sh_ref, o_ref):
    w0 = w_ref[:, 0:1]
    w1 = w_ref[:, 1:2]
    o_ref[...] = w0 * y0_ref[...] + w1 * y1_ref[...] + sh_ref[...]


def _final_combine(w, y0, y1, shared, interpret=False):
    nt = N // BLK
    return pl.pallas_call(
        _final_body,
        grid=(nt,),
        in_specs=[
            pl.BlockSpec((BLK, 2), lambda t: (t, 0)),
            pl.BlockSpec((BLK, D), lambda t: (t, 0)),
            pl.BlockSpec((BLK, D), lambda t: (t, 0)),
            pl.BlockSpec((BLK, D), lambda t: (t, 0)),
        ],
        out_specs=pl.BlockSpec((BLK, D), lambda t: (t, 0)),
        out_shape=jax.ShapeDtypeStruct((N, D), jnp.float32),
        interpret=interpret,
    )(w, y0, y1, shared)


def kernel(x, router_w, expert_gate, expert_up, expert_down,
           shared_gate, shared_up, shared_down):
    xf = x.reshape(N, D)
    p0r, p1r, w, te, va = _router_dispatch(xf, router_w.T)
    p0 = p0r.reshape(N)
    p1 = p1r.reshape(N)
    xg = _sc_dispatch(xf, p0, p1)
    shared = _shared_ffn(xf, shared_gate, shared_up, shared_down)
    yg = _grouped_ffn(te, va, xg, expert_gate, expert_up, expert_down)
    y0, y1 = _sc_combine(yg, p0, p1)
    out = _final_combine(w, y0, y1, shared)
    return out.reshape(1, N, D)
